# baseline (speedup 1.0000x reference)
import jax
import jax.numpy as jnp
from jax.experimental import pallas as pl
from jax.experimental.pallas import tpu as pltpu

_BUCKET = 128


def _sinkhorn_attn_kernel(q_ref, k_ref, v_ref, o_ref):
    t, dh = q_ref.shape[1], q_ref.shape[2]
    nb = t // _BUCKET
    scale = dh ** -0.5

    sq = jnp.concatenate(
        [jnp.mean(q_ref[0, u * _BUCKET:(u + 1) * _BUCKET, :], axis=0,
                  keepdims=True) for u in range(nb)], axis=0)
    sk = jnp.concatenate(
        [jnp.mean(k_ref[0, u * _BUCKET:(u + 1) * _BUCKET, :], axis=0,
                  keepdims=True) for u in range(nb)], axis=0)
    r = jax.lax.dot_general(sq, sk, (((1,), (1,)), ((), ())),
                            preferred_element_type=jnp.float32) * scale

    iota_row = jax.lax.broadcasted_iota(jnp.int32, (1, nb), 1)
    bf = jnp.bfloat16

    for u in range(nb):
        row = jax.lax.slice(r, (u, 0), (u + 1, nb))
        m = jnp.max(row)
        w_u = 1.0 / jnp.sum(jnp.exp(row - m))
        idx_u = jnp.min(jnp.where(row >= m, iota_row, nb))

        qb = q_ref[0, u * _BUCKET:(u + 1) * _BUCKET, :].astype(bf)
        kl = k_ref[0, u * _BUCKET:(u + 1) * _BUCKET, :].astype(bf)
        vl = v_ref[0, u * _BUCKET:(u + 1) * _BUCKET, :].astype(bf)
        kg = k_ref[0, pl.ds(idx_u * _BUCKET, _BUCKET), :].astype(bf)
        vg = v_ref[0, pl.ds(idx_u * _BUCKET, _BUCKET), :].astype(bf)

        s1 = jax.lax.dot_general(qb, kg, (((1,), (1,)), ((), ())),
                                 preferred_element_type=jnp.float32)
        s1 = s1 * (scale * w_u)
        s2 = jax.lax.dot_general(qb, kl, (((1,), (1,)), ((), ())),
                                 preferred_element_type=jnp.float32) * scale
        smax = jnp.maximum(jnp.max(s1, axis=1, keepdims=True),
                           jnp.max(s2, axis=1, keepdims=True))
        e1 = jnp.exp(s1 - smax)
        e2 = jnp.exp(s2 - smax)
        den = (jnp.sum(e1, axis=1, keepdims=True)
               + jnp.sum(e2, axis=1, keepdims=True))
        o1 = jax.lax.dot_general(e1.astype(bf), vg, (((1,), (0,)), ((), ())),
                                 preferred_element_type=jnp.float32)
        o2 = jax.lax.dot_general(e2.astype(bf), vl, (((1,), (0,)), ((), ())),
                                 preferred_element_type=jnp.float32)
        o_ref[0, u * _BUCKET:(u + 1) * _BUCKET, :] = (o1 * w_u + o2) / den


def kernel(q, k, v):
    b, h, t, dh = q.shape
    bh = b * h
    qm = q.reshape(bh, t, dh)
    km = k.reshape(bh, t, dh)
    vm = v.reshape(bh, t, dh)
    out = pl.pallas_call(
        _sinkhorn_attn_kernel,
        grid=(bh,),
        in_specs=[
            pl.BlockSpec((1, t, dh), lambda i: (i, 0, 0)),
            pl.BlockSpec((1, t, dh), lambda i: (i, 0, 0)),
            pl.BlockSpec((1, t, dh), lambda i: (i, 0, 0)),
        ],
        out_specs=pl.BlockSpec((1, t, dh), lambda i: (i, 0, 0)),
        out_shape=jax.ShapeDtypeStruct((bh, t, dh), q.dtype),
        compiler_params=pltpu.CompilerParams(
            dimension_semantics=("arbitrary",)),
    )(qm, km, vm)
    return out.reshape(b, h, t, dh)


# speedup vs baseline: 1.6591x; 1.6591x over previous
import jax
import jax.numpy as jnp
from jax.experimental import pallas as pl
from jax.experimental.pallas import tpu as pltpu

_BUCKET = 128


def _router_kernel(q_ref, k_ref, idx_ref, w_ref):
    t, dh = q_ref.shape[1], q_ref.shape[2]
    nb = t // _BUCKET
    scale = dh ** -0.5

    sq = jnp.concatenate(
        [jnp.mean(q_ref[0, u * _BUCKET:(u + 1) * _BUCKET, :], axis=0,
                  keepdims=True) for u in range(nb)], axis=0)
    sk = jnp.concatenate(
        [jnp.mean(k_ref[0, u * _BUCKET:(u + 1) * _BUCKET, :], axis=0,
                  keepdims=True) for u in range(nb)], axis=0)
    r = jax.lax.dot_general(sq, sk, (((1,), (1,)), ((), ())),
                            preferred_element_type=jnp.float32) * scale
    rmax = jnp.max(r, axis=1, keepdims=True)
    w = 1.0 / jnp.sum(jnp.exp(r - rmax), axis=1, keepdims=True)
    iota = jax.lax.broadcasted_iota(jnp.int32, (nb, nb), 1)
    idx = jnp.min(jnp.where(r >= rmax, iota, nb), axis=1, keepdims=True)
    idx_ref[0] = idx
    w_ref[0] = w


def _attn_kernel(idx_sref, w_sref, q_ref, k_ref, v_ref, o_ref):
    t, dh = q_ref.shape[1], q_ref.shape[2]
    nb = t // _BUCKET
    scale = dh ** -0.5
    i = pl.program_id(0)

    for u in range(nb):
        idx_u = idx_sref[i, u]
        w_u = w_sref[i, u]

        qb = q_ref[0, u * _BUCKET:(u + 1) * _BUCKET, :] * scale
        kl = k_ref[0, u * _BUCKET:(u + 1) * _BUCKET, :]
        vl = v_ref[0, u * _BUCKET:(u + 1) * _BUCKET, :]
        kg = k_ref[0, pl.ds(idx_u * _BUCKET, _BUCKET), :]
        vg = v_ref[0, pl.ds(idx_u * _BUCKET, _BUCKET), :]

        kcat = jnp.concatenate([kg * w_u, kl], axis=0)
        vcat = jnp.concatenate([vg * w_u, vl], axis=0)
        s = jax.lax.dot_general(qb, kcat, (((1,), (1,)), ((), ())),
                                preferred_element_type=jnp.float32)
        smax = jnp.max(s, axis=1, keepdims=True)
        p = jnp.exp(s - smax)
        den = jnp.sum(p, axis=1, keepdims=True)
        o = jax.lax.dot_general(p, vcat, (((1,), (0,)), ((), ())),
                                preferred_element_type=jnp.float32)
        o_ref[0, u * _BUCKET:(u + 1) * _BUCKET, :] = o / den


def kernel(q, k, v):
    b, h, t, dh = q.shape
    bh = b * h
    nb = t // _BUCKET
    qm = q.reshape(bh, t, dh)
    km = k.reshape(bh, t, dh)
    vm = v.reshape(bh, t, dh)

    idx3, w3 = pl.pallas_call(
        _router_kernel,
        grid=(bh,),
        in_specs=[
            pl.BlockSpec((1, t, dh), lambda i: (i, 0, 0)),
            pl.BlockSpec((1, t, dh), lambda i: (i, 0, 0)),
        ],
        out_specs=[
            pl.BlockSpec((1, nb, 1), lambda i: (i, 0, 0)),
            pl.BlockSpec((1, nb, 1), lambda i: (i, 0, 0)),
        ],
        out_shape=[
            jax.ShapeDtypeStruct((bh, nb, 1), jnp.int32),
            jax.ShapeDtypeStruct((bh, nb, 1), jnp.float32),
        ],
        compiler_params=pltpu.CompilerParams(
            dimension_semantics=("arbitrary",)),
    )(qm, km)
    idx = idx3.reshape(bh, nb)
    w = w3.reshape(bh, nb)

    grid_spec = pltpu.PrefetchScalarGridSpec(
        num_scalar_prefetch=2,
        grid=(bh,),
        in_specs=[
            pl.BlockSpec((1, t, dh), lambda i, *_: (i, 0, 0)),
            pl.BlockSpec((1, t, dh), lambda i, *_: (i, 0, 0)),
            pl.BlockSpec((1, t, dh), lambda i, *_: (i, 0, 0)),
        ],
        out_specs=pl.BlockSpec((1, t, dh), lambda i, *_: (i, 0, 0)),
    )
    out = pl.pallas_call(
        _attn_kernel,
        grid_spec=grid_spec,
        out_shape=jax.ShapeDtypeStruct((bh, t, dh), q.dtype),
        compiler_params=pltpu.CompilerParams(
            dimension_semantics=("arbitrary",)),
    )(idx, w, qm, km, vm)
    return out.reshape(b, h, t, dh)
